# Initial kernel scaffold; baseline (speedup 1.0000x reference)
import functools

import jax
import jax.numpy as jnp
from jax import lax
from jax.experimental import pallas as pl
from jax.experimental.pallas import tpu as pltpu
from jax.experimental.pallas import tpu_sc as plsc

_NC = 2
_NS = 16
_NW = _NC * _NS
_L = 16

_N = 10000
_E = 320000
_D = 128

_K = 80
_CHUNKS = _E // _NW // _K
_RPS = _N // _NS
_ZR = 125

_ROW_BLK = 1250


def _dot(a, b):
    return lax.dot_general(a, b, (((1,), (0,)), ((), ())),
                           precision=lax.Precision.HIGHEST,
                           preferred_element_type=jnp.float32)


def _dense_pre(x, W_l, W_r, b):
    def body(x_ref, wl_ref, wr_ref, b_ref, p_ref, r_ref):
        xb = x_ref[...]
        p_ref[...] = _dot(xb, wl_ref[...])
        r_ref[...] = _dot(xb, wr_ref[...]) + b_ref[...]

    n = x.shape[0]
    grid = (n // _ROW_BLK,)
    return pl.pallas_call(
        body,
        grid=grid,
        in_specs=[
            pl.BlockSpec((_ROW_BLK, _D), lambda i: (i, 0)),
            pl.BlockSpec((_D, _D), lambda i: (0, 0)),
            pl.BlockSpec((_D, _D), lambda i: (0, 0)),
            pl.BlockSpec((1, _D), lambda i: (0, 0)),
        ],
        out_specs=[pl.BlockSpec((_ROW_BLK, _D), lambda i: (i, 0))] * 2,
        out_shape=[jax.ShapeDtypeStruct((n, _D), jnp.float32)] * 2,
    )(x, W_l, W_r, b.reshape(1, _D))


def _dense_mid(agg_a, agg_b, cnt_a, cnt_b, r1, W_l, W_r, b):
    def body(aa_ref, ab_ref, ca_ref, cb_ref, r1_ref, wl_ref, wr_ref, b_ref,
             p_ref, r_ref):
        cnt = ca_ref[...][:, 0:1] + cb_ref[...][:, 0:1]
        inv = 1.0 / jnp.maximum(cnt, 1.0)
        h = jnp.maximum((aa_ref[...] + ab_ref[...]) * inv + r1_ref[...], 0.0)
        p_ref[...] = _dot(h, wl_ref[...])
        r_ref[...] = _dot(h, wr_ref[...]) + b_ref[...]

    n = agg_a.shape[0]
    grid = (n // _ROW_BLK,)
    row = pl.BlockSpec((_ROW_BLK, _D), lambda i: (i, 0))
    cntspec = pl.BlockSpec((_ROW_BLK, _L), lambda i: (i, 0))
    wspec = pl.BlockSpec((_D, _D), lambda i: (0, 0))
    return pl.pallas_call(
        body,
        grid=grid,
        in_specs=[row, row, cntspec, cntspec, row, wspec, wspec,
                  pl.BlockSpec((1, _D), lambda i: (0, 0))],
        out_specs=[row, row],
        out_shape=[jax.ShapeDtypeStruct((n, _D), jnp.float32)] * 2,
    )(agg_a, agg_b, cnt_a, cnt_b, r1, W_l, W_r, b.reshape(1, _D))


def _dense_post(agg_a, agg_b, cnt_a, cnt_b, r2):
    def body(aa_ref, ab_ref, ca_ref, cb_ref, r2_ref, o_ref):
        cnt = ca_ref[...][:, 0:1] + cb_ref[...][:, 0:1]
        inv = 1.0 / jnp.maximum(cnt, 1.0)
        o_ref[...] = (aa_ref[...] + ab_ref[...]) * inv + r2_ref[...]

    n = agg_a.shape[0]
    grid = (n // _ROW_BLK,)
    row = pl.BlockSpec((_ROW_BLK, _D), lambda i: (i, 0))
    cntspec = pl.BlockSpec((_ROW_BLK, _L), lambda i: (i, 0))
    return pl.pallas_call(
        body,
        grid=grid,
        in_specs=[row, row, cntspec, cntspec, row],
        out_specs=row,
        out_shape=jax.ShapeDtypeStruct((n, _D), jnp.float32),
    )(agg_a, agg_b, cnt_a, cnt_b, r2)


def _make_seg_sum(with_count):
    mesh = plsc.VectorSubcoreMesh(core_axis_name="c", subcore_axis_name="s")

    out_type = [jax.ShapeDtypeStruct((_NC, _N, _D), jnp.float32)]
    scratch = [
        pltpu.VMEM((_CHUNKS, _K), jnp.int32),
        pltpu.VMEM((_CHUNKS, _K), jnp.int32),
        pltpu.VMEM((_K, _D), jnp.float32),
        pltpu.VMEM((_ZR, _D), jnp.float32),
        pltpu.VMEM_SHARED((_N, _D), jnp.float32),
    ]
    if with_count:
        out_type.append(jax.ShapeDtypeStruct((_NC, _N, _L), jnp.float32))
        scratch += [
            pltpu.VMEM((_K, _L), jnp.float32),
            pltpu.VMEM((_ZR, _L), jnp.float32),
            pltpu.VMEM_SHARED((_N, _L), jnp.float32),
        ]

    def body(table_hbm, src_hbm, dst_hbm, *refs):
        if with_count:
            (acc_out, cnt_out, src_v, dst_v, rows_v, zero_v, acc_s,
             ones_v, zero16_v, cnt_s) = refs
        else:
            acc_out, src_v, dst_v, rows_v, zero_v, acc_s = refs

        cid = lax.axis_index("c")
        sid = lax.axis_index("s")
        wid = sid * _NC + cid

        @pl.loop(0, _ZR)
        def _(i):
            for c in range(_D // _L):
                zero_v.at[pl.ds(i, 1), pl.ds(c * _L, _L)][...] = (
                    jnp.zeros((1, _L), jnp.float32))
            if with_count:
                zero16_v.at[pl.ds(i, 1), pl.ds(0, _L)][...] = (
                    jnp.zeros((1, _L), jnp.float32))

        if with_count:
            @pl.loop(0, _K)
            def _(i):
                ones_v.at[pl.ds(i, 1), pl.ds(0, _L)][...] = (
                    jnp.ones((1, _L), jnp.float32))

        for blk in range(_RPS // _ZR):
            base = sid * _RPS + blk * _ZR
            pltpu.sync_copy(zero_v, acc_s.at[pl.ds(base, _ZR), :])
            if with_count:
                pltpu.sync_copy(zero16_v, cnt_s.at[pl.ds(base, _ZR), :])
        plsc.subcore_barrier()

        pltpu.sync_copy(src_hbm.at[wid], src_v)
        pltpu.sync_copy(dst_hbm.at[wid], dst_v)

        @pl.loop(0, _CHUNKS)
        def _(j):
            pltpu.sync_copy(table_hbm.at[src_v.at[j]], rows_v)
            pltpu.sync_copy(rows_v, acc_s.at[dst_v.at[j]], add=True)
            if with_count:
                pltpu.sync_copy(ones_v, cnt_s.at[dst_v.at[j]], add=True)

        plsc.subcore_barrier()

        rows = pl.ds(sid * _RPS, _RPS)
        pltpu.sync_copy(acc_s.at[rows, :], acc_out.at[cid, rows, :])
        if with_count:
            pltpu.sync_copy(cnt_s.at[rows, :], cnt_out.at[cid, rows, :])

    return functools.partial(pl.kernel, mesh=mesh, out_type=out_type,
                             scratch_types=scratch)(body)


_seg_sum_count = _make_seg_sum(with_count=True)
_seg_sum = _make_seg_sum(with_count=False)


def kernel(x, edge_index, W_l1, b1, W_r1, W_l2, b2, W_r2):
    src = edge_index[0].reshape(_NW, _CHUNKS, _K)
    dst = edge_index[1].reshape(_NW, _CHUNKS, _K)

    p1, r1 = _dense_pre(x, W_l1, W_r1, b1)
    agg1, cnt = _seg_sum_count(p1, src, dst)
    p2, r2 = _dense_mid(agg1[0], agg1[1], cnt[0], cnt[1], r1, W_l2, W_r2, b2)
    agg2 = _seg_sum(p2, src, dst)
    return _dense_post(agg2[0], agg2[1], cnt[0], cnt[1], r2)


# speedup vs baseline: 4.9183x; 4.9183x over previous
import functools

import jax
import jax.numpy as jnp
from jax import lax
from jax.experimental import pallas as pl
from jax.experimental.pallas import tpu as pltpu
from jax.experimental.pallas import tpu_sc as plsc

_NC = 2
_NS = 16
_NW = _NC * _NS
_L = 16

_N = 10000
_E = 320000
_D = 128
_DH = _D // 2

_NP = 10240
_K = 80
_CHUNKS = _E // _NW // _K
_RPS = _NP // _NS
_ZR = 128

_ROW_BLK = 1000


def _dot(a, b):
    return lax.dot_general(a, b, (((1,), (0,)), ((), ())),
                           precision=lax.Precision.HIGHEST,
                           preferred_element_type=jnp.float32)


def _dense_pre(x, W_l, W_r, b):
    def body(x_ref, wl_ref, wr_ref, b_ref, pa_ref, pb_ref, r_ref):
        xb = x_ref[...]
        p = _dot(xb, wl_ref[...])
        pa_ref[...] = p[:, :_DH]
        pb_ref[...] = p[:, _DH:]
        r_ref[...] = _dot(xb, wr_ref[...]) + b_ref[...]

    n = _N
    grid = (n // _ROW_BLK,)
    half = pl.BlockSpec((_ROW_BLK, _DH), lambda i: (i, 0))
    row = pl.BlockSpec((_ROW_BLK, _D), lambda i: (i, 0))
    return pl.pallas_call(
        body,
        grid=grid,
        in_specs=[
            row,
            pl.BlockSpec((_D, _D), lambda i: (0, 0)),
            pl.BlockSpec((_D, _D), lambda i: (0, 0)),
            pl.BlockSpec((1, _D), lambda i: (0, 0)),
        ],
        out_specs=[half, half, row],
        out_shape=[jax.ShapeDtypeStruct((n, _DH), jnp.float32)] * 2
        + [jax.ShapeDtypeStruct((n, _D), jnp.float32)],
    )(x, W_l, W_r, b.reshape(1, _D))


def _merge_agg(aa0, aa1, ab0, ab1, ca, cb, r):
    cnt = ca[:, 0:1] + cb[:, 0:1]
    inv = 1.0 / jnp.maximum(cnt, 1.0)
    ha = (aa0 + aa1) * inv + r[:, :_DH]
    hb = (ab0 + ab1) * inv + r[:, _DH:]
    return jnp.concatenate([ha, hb], axis=1)


def _dense_mid(aa, ab, cnt, r1, W_l, W_r, b):
    def body(aa0_ref, aa1_ref, ab0_ref, ab1_ref, ca_ref, cb_ref, r1_ref,
             wl_ref, wr_ref, b_ref, pa_ref, pb_ref, r_ref):
        h = jnp.maximum(
            _merge_agg(aa0_ref[...], aa1_ref[...], ab0_ref[...], ab1_ref[...],
                       ca_ref[...], cb_ref[...], r1_ref[...]), 0.0)
        p = _dot(h, wl_ref[...])
        pa_ref[...] = p[:, :_DH]
        pb_ref[...] = p[:, _DH:]
        r_ref[...] = _dot(h, wr_ref[...]) + b_ref[...]

    n = _N
    grid = (n // _ROW_BLK,)
    half = pl.BlockSpec((_ROW_BLK, _DH), lambda i: (i, 0))
    row = pl.BlockSpec((_ROW_BLK, _D), lambda i: (i, 0))
    wspec = pl.BlockSpec((_D, _D), lambda i: (0, 0))
    return pl.pallas_call(
        body,
        grid=grid,
        in_specs=[half, half, half, half, half, half, row,
                  wspec, wspec, pl.BlockSpec((1, _D), lambda i: (0, 0))],
        out_specs=[half, half, row],
        out_shape=[jax.ShapeDtypeStruct((n, _DH), jnp.float32)] * 2
        + [jax.ShapeDtypeStruct((n, _D), jnp.float32)],
    )(aa[0], aa[1], ab[0], ab[1], cnt[0], cnt[1], r1, W_l, W_r,
      b.reshape(1, _D))


def _dense_post(aa, ab, cnt, r2):
    def body(aa0_ref, aa1_ref, ab0_ref, ab1_ref, ca_ref, cb_ref, r2_ref,
             o_ref):
        o_ref[...] = _merge_agg(
            aa0_ref[...], aa1_ref[...], ab0_ref[...], ab1_ref[...],
            ca_ref[...], cb_ref[...], r2_ref[...])

    n = _N
    grid = (n // _ROW_BLK,)
    half = pl.BlockSpec((_ROW_BLK, _DH), lambda i: (i, 0))
    row = pl.BlockSpec((_ROW_BLK, _D), lambda i: (i, 0))
    return pl.pallas_call(
        body,
        grid=grid,
        in_specs=[half, half, half, half, half, half, row],
        out_specs=row,
        out_shape=jax.ShapeDtypeStruct((n, _D), jnp.float32),
    )(aa[0], aa[1], ab[0], ab[1], cnt[0], cnt[1], r2)


_sc_mesh = plsc.VectorSubcoreMesh(core_axis_name="c", subcore_axis_name="s")
_sc_params = pltpu.CompilerParams(use_tc_tiling_on_sc=False)


def _make_seg_sum(with_count):
    out_type = [jax.ShapeDtypeStruct((_NC, _NP, _DH), jnp.float32)] * (
        3 if with_count else 2)
    scratch = [
        pltpu.VMEM((_CHUNKS, _K), jnp.int32),
        pltpu.VMEM((_CHUNKS, _K), jnp.int32),
        pltpu.VMEM((_K, _DH), jnp.float32),
        pltpu.VMEM((_ZR, _DH), jnp.float32),
        pltpu.VMEM_SHARED((_NP, _DH), jnp.float32),
    ]

    def body(ta_hbm, tb_hbm, src_hbm, dst_hbm, *refs):
        if with_count:
            outa, outb, outc, src_v, dst_v, rows_v, zero_v, acc_s = refs
        else:
            outa, outb, src_v, dst_v, rows_v, zero_v, acc_s = refs

        cid = lax.axis_index("c")
        sid = lax.axis_index("s")
        wid = sid * _NC + cid
        stripe = pl.ds(sid * _RPS, _RPS)

        @pl.loop(0, _ZR)
        def _(i):
            for c in range(_DH // _L):
                zero_v.at[pl.ds(i, 1), pl.ds(c * _L, _L)][...] = (
                    jnp.zeros((1, _L), jnp.float32))

        def zero_stripe():
            for blk in range(_RPS // _ZR):
                base = sid * _RPS + blk * _ZR
                pltpu.sync_copy(zero_v, acc_s.at[pl.ds(base, _ZR), :])

        zero_stripe()

        pltpu.sync_copy(src_hbm.at[wid], src_v)
        pltpu.sync_copy(dst_hbm.at[wid], dst_v)
        plsc.subcore_barrier()

        passes = [(ta_hbm, outa), (tb_hbm, outb)]
        if with_count:
            passes.append((None, outc))

        for pi, (table, out) in enumerate(passes):
            if table is None:
                @pl.loop(0, _K)
                def _(i):
                    for c in range(_DH // _L):
                        rows_v.at[pl.ds(i, 1), pl.ds(c * _L, _L)][...] = (
                            jnp.ones((1, _L), jnp.float32))

                @pl.loop(0, _CHUNKS)
                def _(j):
                    pltpu.sync_copy(rows_v, acc_s.at[dst_v.at[j]], add=True)
            else:
                @pl.loop(0, _CHUNKS)
                def _(j):
                    pltpu.sync_copy(table.at[src_v.at[j]], rows_v)
                    pltpu.sync_copy(rows_v, acc_s.at[dst_v.at[j]], add=True)

            plsc.subcore_barrier()
            pltpu.sync_copy(acc_s.at[stripe, :], out.at[cid, stripe, :])
            plsc.subcore_barrier()
            if pi + 1 < len(passes):
                zero_stripe()
                plsc.subcore_barrier()

    return functools.partial(pl.kernel, mesh=_sc_mesh, out_type=out_type,
                             scratch_types=scratch,
                             compiler_params=_sc_params)(body)


_seg_sum_count = _make_seg_sum(with_count=True)
_seg_sum = _make_seg_sum(with_count=False)


def kernel(x, edge_index, W_l1, b1, W_r1, W_l2, b2, W_r2):
    src = edge_index[0].reshape(_NW, _CHUNKS, _K)
    dst = edge_index[1].reshape(_NW, _CHUNKS, _K)

    p1a, p1b, r1 = _dense_pre(x, W_l1, W_r1, b1)
    agg1a, agg1b, cnt = _seg_sum_count(p1a, p1b, src, dst)
    p2a, p2b, r2 = _dense_mid(agg1a, agg1b, cnt, r1, W_l2, W_r2, b2)
    agg2a, agg2b = _seg_sum(p2a, p2b, src, dst)
    return _dense_post(agg2a, agg2b, cnt, r2)


# speedup vs baseline: 9.4970x; 1.9309x over previous
import functools

import jax
import jax.numpy as jnp
from jax import lax
from jax.experimental import pallas as pl
from jax.experimental.pallas import tpu as pltpu
from jax.experimental.pallas import tpu_sc as plsc

_NC = 2
_NS = 16
_NW = _NC * _NS
_L = 16

_N = 10000
_E = 320000
_D = 128
_DH = _D // 2

_NP = 10240
_K = 80
_CHUNKS = _E // _NW // _K
_NBUF = 5
_GRP = _CHUNKS // _NBUF
_RPS = _NP // _NS
_ZR = 128

_ROW_BLK = 1000


def _dot(a, b):
    return lax.dot_general(a, b, (((1,), (0,)), ((), ())),
                           precision=lax.Precision.HIGHEST,
                           preferred_element_type=jnp.float32)


def _dense_pre(x, W_l, W_r, b):
    def body(x_ref, wl_ref, wr_ref, b_ref, pa_ref, pb_ref, r_ref):
        xb = x_ref[...]
        p = _dot(xb, wl_ref[...])
        pa_ref[...] = p[:, :_DH]
        pb_ref[...] = p[:, _DH:]
        r_ref[...] = _dot(xb, wr_ref[...]) + b_ref[...]

    n = _N
    grid = (n // _ROW_BLK,)
    half = pl.BlockSpec((_ROW_BLK, _DH), lambda i: (i, 0))
    row = pl.BlockSpec((_ROW_BLK, _D), lambda i: (i, 0))
    return pl.pallas_call(
        body,
        grid=grid,
        in_specs=[
            row,
            pl.BlockSpec((_D, _D), lambda i: (0, 0)),
            pl.BlockSpec((_D, _D), lambda i: (0, 0)),
            pl.BlockSpec((1, _D), lambda i: (0, 0)),
        ],
        out_specs=[half, half, row],
        out_shape=[jax.ShapeDtypeStruct((n, _DH), jnp.float32)] * 2
        + [jax.ShapeDtypeStruct((n, _D), jnp.float32)],
    )(x, W_l, W_r, b.reshape(1, _D))


def _merge_agg(aa0, aa1, ab0, ab1, ca, cb, r):
    cnt = ca[:, 0:1] + cb[:, 0:1]
    inv = 1.0 / jnp.maximum(cnt, 1.0)
    ha = (aa0 + aa1) * inv + r[:, :_DH]
    hb = (ab0 + ab1) * inv + r[:, _DH:]
    return jnp.concatenate([ha, hb], axis=1)


def _dense_mid(aa, ab, cnt, r1, W_l, W_r, b):
    def body(aa0_ref, aa1_ref, ab0_ref, ab1_ref, ca_ref, cb_ref, r1_ref,
             wl_ref, wr_ref, b_ref, pa_ref, pb_ref, r_ref):
        h = jnp.maximum(
            _merge_agg(aa0_ref[...], aa1_ref[...], ab0_ref[...], ab1_ref[...],
                       ca_ref[...], cb_ref[...], r1_ref[...]), 0.0)
        p = _dot(h, wl_ref[...])
        pa_ref[...] = p[:, :_DH]
        pb_ref[...] = p[:, _DH:]
        r_ref[...] = _dot(h, wr_ref[...]) + b_ref[...]

    n = _N
    grid = (n // _ROW_BLK,)
    half = pl.BlockSpec((_ROW_BLK, _DH), lambda i: (i, 0))
    row = pl.BlockSpec((_ROW_BLK, _D), lambda i: (i, 0))
    wspec = pl.BlockSpec((_D, _D), lambda i: (0, 0))
    return pl.pallas_call(
        body,
        grid=grid,
        in_specs=[half, half, half, half, half, half, row,
                  wspec, wspec, pl.BlockSpec((1, _D), lambda i: (0, 0))],
        out_specs=[half, half, row],
        out_shape=[jax.ShapeDtypeStruct((n, _DH), jnp.float32)] * 2
        + [jax.ShapeDtypeStruct((n, _D), jnp.float32)],
    )(aa[0], aa[1], ab[0], ab[1], cnt[0], cnt[1], r1, W_l, W_r,
      b.reshape(1, _D))


def _dense_post(aa, ab, cnt, r2):
    def body(aa0_ref, aa1_ref, ab0_ref, ab1_ref, ca_ref, cb_ref, r2_ref,
             o_ref):
        o_ref[...] = _merge_agg(
            aa0_ref[...], aa1_ref[...], ab0_ref[...], ab1_ref[...],
            ca_ref[...], cb_ref[...], r2_ref[...])

    n = _N
    grid = (n // _ROW_BLK,)
    half = pl.BlockSpec((_ROW_BLK, _DH), lambda i: (i, 0))
    row = pl.BlockSpec((_ROW_BLK, _D), lambda i: (i, 0))
    return pl.pallas_call(
        body,
        grid=grid,
        in_specs=[half, half, half, half, half, half, row],
        out_specs=row,
        out_shape=jax.ShapeDtypeStruct((n, _D), jnp.float32),
    )(aa[0], aa[1], ab[0], ab[1], cnt[0], cnt[1], r2)


_sc_mesh = plsc.VectorSubcoreMesh(core_axis_name="c", subcore_axis_name="s")
_sc_params = pltpu.CompilerParams(use_tc_tiling_on_sc=False)


def _make_seg_sum(with_count):
    out_type = [jax.ShapeDtypeStruct((_NC, _NP, _DH), jnp.float32)] * (
        3 if with_count else 2)
    scratch = [
        pltpu.VMEM((_CHUNKS, _K), jnp.int32),
        pltpu.VMEM((_CHUNKS, _K), jnp.int32),
        pltpu.VMEM((_NBUF, _K, _DH), jnp.float32),
        pltpu.VMEM((_ZR, _DH), jnp.float32),
        pltpu.VMEM_SHARED((_NP, _DH), jnp.float32),
    ] + [pltpu.SemaphoreType.DMA] * (2 * _NBUF)

    def body(ta_hbm, tb_hbm, src_hbm, dst_hbm, *refs):
        if with_count:
            (outa, outb, outc, src_v, dst_v, rows_v, zero_v, acc_s,
             *sems) = refs
        else:
            outa, outb, src_v, dst_v, rows_v, zero_v, acc_s, *sems = refs
        gsem, ssem = sems[:_NBUF], sems[_NBUF:]

        cid = lax.axis_index("c")
        sid = lax.axis_index("s")
        wid = sid * _NC + cid
        stripe = pl.ds(sid * _RPS, _RPS)

        @pl.loop(0, _ZR)
        def _(i):
            for c in range(_DH // _L):
                zero_v.at[pl.ds(i, 1), pl.ds(c * _L, _L)][...] = (
                    jnp.zeros((1, _L), jnp.float32))

        def zero_stripe():
            for blk in range(_RPS // _ZR):
                base = sid * _RPS + blk * _ZR
                pltpu.sync_copy(zero_v, acc_s.at[pl.ds(base, _ZR), :])

        def wait_gather(table, b):
            pltpu.make_async_copy(
                table.at[src_v.at[b]], rows_v.at[b], gsem[b]).wait()

        def wait_scatter(b):
            pltpu.make_async_copy(
                rows_v.at[b], acc_s.at[dst_v.at[b]], ssem[b]).wait()

        def data_pass(table):
            for b in range(_NBUF):
                pltpu.async_copy(table.at[src_v.at[b]], rows_v.at[b],
                                 gsem[b])

            @pl.loop(0, _GRP)
            def _(g):
                for b in range(_NBUF):
                    j = g * _NBUF + b
                    wait_gather(table, b)
                    pltpu.async_copy(rows_v.at[b], acc_s.at[dst_v.at[j]],
                                     ssem[b], add=True)

                    @pl.when(g < _GRP - 1)
                    def _():
                        wait_scatter(b)
                        pltpu.async_copy(table.at[src_v.at[j + _NBUF]],
                                         rows_v.at[b], gsem[b])

            for b in range(_NBUF):
                wait_scatter(b)

        def count_pass():
            @pl.loop(0, _K)
            def _(i):
                for c in range(_DH // _L):
                    rows_v.at[pl.ds(0, 1), pl.ds(i, 1),
                              pl.ds(c * _L, _L)][...] = (
                        jnp.ones((1, 1, _L), jnp.float32))

            @pl.loop(0, _GRP)
            def _(g):
                for b in range(_NBUF):
                    j = g * _NBUF + b

                    @pl.when(g > 0)
                    def _():
                        pltpu.make_async_copy(
                            rows_v.at[0], acc_s.at[dst_v.at[b]],
                            ssem[b]).wait()

                    pltpu.async_copy(rows_v.at[0], acc_s.at[dst_v.at[j]],
                                     ssem[b], add=True)

            for b in range(_NBUF):
                pltpu.make_async_copy(
                    rows_v.at[0], acc_s.at[dst_v.at[b]], ssem[b]).wait()

        zero_stripe()

        pltpu.sync_copy(src_hbm.at[wid], src_v)
        pltpu.sync_copy(dst_hbm.at[wid], dst_v)
        plsc.subcore_barrier()

        passes = [(ta_hbm, outa), (tb_hbm, outb)]
        if with_count:
            passes.append((None, outc))

        for pi, (table, out) in enumerate(passes):
            if table is None:
                count_pass()
            else:
                data_pass(table)

            plsc.subcore_barrier()
            pltpu.sync_copy(acc_s.at[stripe, :], out.at[cid, stripe, :])
            plsc.subcore_barrier()
            if pi + 1 < len(passes):
                zero_stripe()
                plsc.subcore_barrier()

    return functools.partial(pl.kernel, mesh=_sc_mesh, out_type=out_type,
                             scratch_types=scratch,
                             compiler_params=_sc_params)(body)


_seg_sum_count = _make_seg_sum(with_count=True)
_seg_sum = _make_seg_sum(with_count=False)


def kernel(x, edge_index, W_l1, b1, W_r1, W_l2, b2, W_r2):
    src = edge_index[0].reshape(_NW, _CHUNKS, _K)
    dst = edge_index[1].reshape(_NW, _CHUNKS, _K)

    p1a, p1b, r1 = _dense_pre(x, W_l1, W_r1, b1)
    agg1a, agg1b, cnt = _seg_sum_count(p1a, p1b, src, dst)
    p2a, p2b, r2 = _dense_mid(agg1a, agg1b, cnt, r1, W_l2, W_r2, b2)
    agg2a, agg2b = _seg_sum(p2a, p2b, src, dst)
    return _dense_post(agg2a, agg2b, cnt, r2)


# speedup vs baseline: 11.4766x; 1.2084x over previous
import functools

import jax
import jax.numpy as jnp
from jax import lax
from jax.experimental import pallas as pl
from jax.experimental.pallas import tpu as pltpu
from jax.experimental.pallas import tpu_sc as plsc

_NC = 2
_NS = 16
_NW = _NC * _NS
_L = 16

_N = 10000
_E = 320000
_D = 128
_DH = _D // 2

_NP = 10240
_K = 80
_CHUNKS = _E // _NW // _K
_NBUF = 5
_GRP = _CHUNKS // _NBUF
_RPS = _NP // _NS
_ZR = 128
_CW = _L

_ROW_BLK = 1000


def _dot(a, b):
    return lax.dot_general(a, b, (((1,), (0,)), ((), ())),
                           precision=lax.Precision.HIGHEST,
                           preferred_element_type=jnp.float32)


def _dense_pre(x, W_l, W_r, b):
    def body(x_ref, wl_ref, wr_ref, b_ref, pa_ref, pb_ref, r_ref):
        xb = x_ref[...]
        p = _dot(xb, wl_ref[...])
        pa_ref[...] = p[:, :_DH]
        pb_ref[...] = p[:, _DH:]
        r_ref[...] = _dot(xb, wr_ref[...]) + b_ref[...]

    n = _N
    grid = (n // _ROW_BLK,)
    half = pl.BlockSpec((_ROW_BLK, _DH), lambda i: (i, 0))
    row = pl.BlockSpec((_ROW_BLK, _D), lambda i: (i, 0))
    return pl.pallas_call(
        body,
        grid=grid,
        in_specs=[
            row,
            pl.BlockSpec((_D, _D), lambda i: (0, 0)),
            pl.BlockSpec((_D, _D), lambda i: (0, 0)),
            pl.BlockSpec((1, _D), lambda i: (0, 0)),
        ],
        out_specs=[half, half, row],
        out_shape=[jax.ShapeDtypeStruct((n, _DH), jnp.float32)] * 2
        + [jax.ShapeDtypeStruct((n, _D), jnp.float32)],
    )(x, W_l, W_r, b.reshape(1, _D))


def _merge_agg(a_ref, b_ref, c_ref, r):
    cnt = c_ref[0, :, 0:1] + c_ref[1, :, 0:1]
    inv = 1.0 / jnp.maximum(cnt, 1.0)
    ha = (a_ref[0] + a_ref[1]) * inv + r[:, :_DH]
    hb = (b_ref[0] + b_ref[1]) * inv + r[:, _DH:]
    return jnp.concatenate([ha, hb], axis=1)


_agg3 = pl.BlockSpec((_NC, _ROW_BLK, _DH), lambda i: (0, i, 0))
_cnt3 = pl.BlockSpec((_NC, _ROW_BLK, _CW), lambda i: (0, i, 0))


def _dense_mid(aa, ab, cnt, r1, W_l, W_r, b):
    def body(aa_ref, ab_ref, c_ref, r1_ref, wl_ref, wr_ref, b_ref,
             pa_ref, pb_ref, r_ref):
        h = jnp.maximum(
            _merge_agg(aa_ref[...], ab_ref[...], c_ref[...], r1_ref[...]),
            0.0)
        p = _dot(h, wl_ref[...])
        pa_ref[...] = p[:, :_DH]
        pb_ref[...] = p[:, _DH:]
        r_ref[...] = _dot(h, wr_ref[...]) + b_ref[...]

    n = _N
    grid = (n // _ROW_BLK,)
    half = pl.BlockSpec((_ROW_BLK, _DH), lambda i: (i, 0))
    row = pl.BlockSpec((_ROW_BLK, _D), lambda i: (i, 0))
    wspec = pl.BlockSpec((_D, _D), lambda i: (0, 0))
    return pl.pallas_call(
        body,
        grid=grid,
        in_specs=[_agg3, _agg3, _cnt3, row,
                  wspec, wspec, pl.BlockSpec((1, _D), lambda i: (0, 0))],
        out_specs=[half, half, row],
        out_shape=[jax.ShapeDtypeStruct((n, _DH), jnp.float32)] * 2
        + [jax.ShapeDtypeStruct((n, _D), jnp.float32)],
    )(aa, ab, cnt, r1, W_l, W_r, b.reshape(1, _D))


def _dense_post(aa, ab, cnt, r2):
    def body(aa_ref, ab_ref, c_ref, r2_ref, o_ref):
        o_ref[...] = _merge_agg(aa_ref[...], ab_ref[...], c_ref[...],
                                r2_ref[...])

    n = _N
    grid = (n // _ROW_BLK,)
    row = pl.BlockSpec((_ROW_BLK, _D), lambda i: (i, 0))
    return pl.pallas_call(
        body,
        grid=grid,
        in_specs=[_agg3, _agg3, _cnt3, row],
        out_specs=row,
        out_shape=jax.ShapeDtypeStruct((n, _D), jnp.float32),
    )(aa, ab, cnt, r2)


_sc_mesh = plsc.VectorSubcoreMesh(core_axis_name="c", subcore_axis_name="s")
_sc_params = pltpu.CompilerParams(use_tc_tiling_on_sc=False)


def _make_seg_sum(with_count):
    out_type = [jax.ShapeDtypeStruct((_NC, _NP, _DH), jnp.float32)] * 2
    scratch = [
        pltpu.VMEM((_CHUNKS, _K), jnp.int32),
        pltpu.VMEM((_CHUNKS, _K), jnp.int32),
        pltpu.VMEM((_NBUF, _K, _DH), jnp.float32),
        pltpu.VMEM((_ZR, _DH), jnp.float32),
        pltpu.VMEM_SHARED((_NP, _DH), jnp.float32),
    ]
    if with_count:
        out_type = out_type + [
            jax.ShapeDtypeStruct((_NC, _NP, _CW), jnp.float32)]
        scratch = scratch + [
            pltpu.VMEM((_K, _CW), jnp.float32),
            pltpu.VMEM_SHARED((_NP, _CW), jnp.float32),
        ]
    scratch = scratch + [pltpu.SemaphoreType.DMA] * (2 * _NBUF)

    def body(ta_hbm, tb_hbm, edges_hbm, *refs):
        if with_count:
            (outa, outb, outc, src_v, dst_v, rows_v, zero_v, acc_s,
             ones_v, accc_s, *sems) = refs
        else:
            outa, outb, src_v, dst_v, rows_v, zero_v, acc_s, *sems = refs
        gsem, ssem = sems[:_NBUF], sems[_NBUF:]

        cid = lax.axis_index("c")
        sid = lax.axis_index("s")
        wid = sid * _NC + cid
        stripe = pl.ds(sid * _RPS, _RPS)

        @pl.loop(0, _ZR)
        def _(i):
            for c in range(_DH // _L):
                zero_v.at[pl.ds(i, 1), pl.ds(c * _L, _L)][...] = (
                    jnp.zeros((1, _L), jnp.float32))

        def zero_stripe():
            for blk in range(_RPS // _ZR):
                base = sid * _RPS + blk * _ZR
                pltpu.sync_copy(zero_v, acc_s.at[pl.ds(base, _ZR), :])

        def wait_gather(table, b):
            pltpu.make_async_copy(
                table.at[src_v.at[b]], rows_v.at[b], gsem[b]).wait()

        def wait_scatter(b):
            pltpu.make_async_copy(
                rows_v.at[b], acc_s.at[dst_v.at[b]], ssem[b]).wait()

        def data_pass(table):
            for b in range(_NBUF):
                pltpu.async_copy(table.at[src_v.at[b]], rows_v.at[b],
                                 gsem[b])

            @pl.loop(0, _GRP)
            def _(g):
                for b in range(_NBUF):
                    j = g * _NBUF + b
                    wait_gather(table, b)
                    pltpu.async_copy(rows_v.at[b], acc_s.at[dst_v.at[j]],
                                     ssem[b], add=True)

                    @pl.when(g < _GRP - 1)
                    def _():
                        wait_scatter(b)
                        pltpu.async_copy(table.at[src_v.at[j + _NBUF]],
                                         rows_v.at[b], gsem[b])

            for b in range(_NBUF):
                wait_scatter(b)

        def count_pass():
            @pl.loop(0, _GRP)
            def _(g):
                for b in range(_NBUF):
                    j = g * _NBUF + b

                    @pl.when(g > 0)
                    def _():
                        pltpu.make_async_copy(
                            ones_v, accc_s.at[dst_v.at[b]], ssem[b]).wait()

                    pltpu.async_copy(ones_v, accc_s.at[dst_v.at[j]],
                                     ssem[b], add=True)

            for b in range(_NBUF):
                pltpu.make_async_copy(
                    ones_v, accc_s.at[dst_v.at[b]], ssem[b]).wait()

        zero_stripe()
        if with_count:
            @pl.loop(0, _K)
            def _(i):
                ones_v.at[pl.ds(i, 1), :][...] = jnp.ones((1, _CW),
                                                          jnp.float32)
            for blk in range(_RPS // _ZR):
                base = sid * _RPS + blk * _ZR
                pltpu.sync_copy(zero_v.at[:, pl.ds(0, _CW)],
                                accc_s.at[pl.ds(base, _ZR), :])

        pltpu.sync_copy(edges_hbm.at[0, wid], src_v)
        pltpu.sync_copy(edges_hbm.at[1, wid], dst_v)
        plsc.subcore_barrier()

        if with_count:
            count_pass()

        passes = [(ta_hbm, outa), (tb_hbm, outb)]
        for pi, (table, out) in enumerate(passes):
            data_pass(table)

            plsc.subcore_barrier()
            pltpu.sync_copy(acc_s.at[stripe, :], out.at[cid, stripe, :])
            if with_count and pi == 0:
                pltpu.sync_copy(accc_s.at[stripe, :],
                                outc.at[cid, stripe, :])
            plsc.subcore_barrier()
            if pi + 1 < len(passes):
                zero_stripe()
                plsc.subcore_barrier()

    return functools.partial(pl.kernel, mesh=_sc_mesh, out_type=out_type,
                             scratch_types=scratch,
                             compiler_params=_sc_params)(body)


_seg_sum_count = _make_seg_sum(with_count=True)
_seg_sum = _make_seg_sum(with_count=False)


def kernel(x, edge_index, W_l1, b1, W_r1, W_l2, b2, W_r2):
    edges = edge_index.reshape(2, _NW, _CHUNKS, _K)

    p1a, p1b, r1 = _dense_pre(x, W_l1, W_r1, b1)
    agg1a, agg1b, cnt = _seg_sum_count(p1a, p1b, edges)
    p2a, p2b, r2 = _dense_mid(agg1a, agg1b, cnt, r1, W_l2, W_r2, b2)
    agg2a, agg2b = _seg_sum(p2a, p2b, edges)
    return _dense_post(agg2a, agg2b, cnt, r2)


# speedup vs baseline: 12.7774x; 1.1133x over previous
import functools

import jax
import jax.numpy as jnp
from jax import lax
from jax.experimental import pallas as pl
from jax.experimental.pallas import tpu as pltpu
from jax.experimental.pallas import tpu_sc as plsc

_NC = 2
_NS = 16
_NW = _NC * _NS
_L = 16

_N = 10000
_E = 320000
_D = 128
_DH = _D // 2

_NP = 10240
_K = 80
_CHUNKS = _E // _NW // _K
_NBUF = 5
_GRP = _CHUNKS // _NBUF
_RPS = _NP // _NS
_ZR = 128
_CW = _L

_ROW_BLK = 1000


def _dot(a, b):
    return lax.dot_general(a, b, (((1,), (0,)), ((), ())),
                           precision=lax.Precision.HIGHEST,
                           preferred_element_type=jnp.float32)


def _dense_pre(x, W_l, W_r, b):
    def body(x_ref, wl_ref, wr_ref, b_ref, pa_ref, pb_ref, r_ref):
        xb = x_ref[...]
        p = _dot(xb, wl_ref[...])
        pa_ref[...] = p[:, :_DH]
        pb_ref[...] = p[:, _DH:]
        r_ref[...] = _dot(xb, wr_ref[...]) + b_ref[...]

    n = _N
    grid = (n // _ROW_BLK,)
    half = pl.BlockSpec((_ROW_BLK, _DH), lambda i: (i, 0))
    row = pl.BlockSpec((_ROW_BLK, _D), lambda i: (i, 0))
    return pl.pallas_call(
        body,
        grid=grid,
        in_specs=[
            row,
            pl.BlockSpec((_D, _D), lambda i: (0, 0)),
            pl.BlockSpec((_D, _D), lambda i: (0, 0)),
            pl.BlockSpec((1, _D), lambda i: (0, 0)),
        ],
        out_specs=[half, half, row],
        out_shape=[jax.ShapeDtypeStruct((n, _DH), jnp.float32)] * 2
        + [jax.ShapeDtypeStruct((n, _D), jnp.float32)],
    )(x, W_l, W_r, b.reshape(1, _D))


def _merge_agg(a_ref, c_ref, r):
    cnt = c_ref[0, :, 0:1] + c_ref[1, :, 0:1]
    inv = 1.0 / jnp.maximum(cnt, 1.0)
    return (a_ref[0] + a_ref[1]) * inv + r


_agg3 = pl.BlockSpec((_NC, _ROW_BLK, _D), lambda i: (0, i, 0))
_cnt3 = pl.BlockSpec((_NC, _ROW_BLK, _CW), lambda i: (0, i, 0))


def _dense_mid(agg, cnt, r1, W_l, W_r, b):
    def body(a_ref, c_ref, r1_ref, wl_ref, wr_ref, b_ref,
             pa_ref, pb_ref, r_ref):
        h = jnp.maximum(_merge_agg(a_ref[...], c_ref[...], r1_ref[...]),
                        0.0)
        p = _dot(h, wl_ref[...])
        pa_ref[...] = p[:, :_DH]
        pb_ref[...] = p[:, _DH:]
        r_ref[...] = _dot(h, wr_ref[...]) + b_ref[...]

    n = _N
    grid = (n // _ROW_BLK,)
    half = pl.BlockSpec((_ROW_BLK, _DH), lambda i: (i, 0))
    row = pl.BlockSpec((_ROW_BLK, _D), lambda i: (i, 0))
    wspec = pl.BlockSpec((_D, _D), lambda i: (0, 0))
    return pl.pallas_call(
        body,
        grid=grid,
        in_specs=[_agg3, _cnt3, row,
                  wspec, wspec, pl.BlockSpec((1, _D), lambda i: (0, 0))],
        out_specs=[half, half, row],
        out_shape=[jax.ShapeDtypeStruct((n, _DH), jnp.float32)] * 2
        + [jax.ShapeDtypeStruct((n, _D), jnp.float32)],
    )(agg, cnt, r1, W_l, W_r, b.reshape(1, _D))


def _dense_post(agg, cnt, r2):
    def body(a_ref, c_ref, r2_ref, o_ref):
        o_ref[...] = _merge_agg(a_ref[...], c_ref[...], r2_ref[...])

    n = _N
    grid = (n // _ROW_BLK,)
    row = pl.BlockSpec((_ROW_BLK, _D), lambda i: (i, 0))
    return pl.pallas_call(
        body,
        grid=grid,
        in_specs=[_agg3, _cnt3, row],
        out_specs=row,
        out_shape=jax.ShapeDtypeStruct((n, _D), jnp.float32),
    )(agg, cnt, r2)


_sc_mesh = plsc.VectorSubcoreMesh(core_axis_name="c", subcore_axis_name="s")
_sc_params = pltpu.CompilerParams(use_tc_tiling_on_sc=False)


def _make_seg_sum(with_count):
    out_type = [jax.ShapeDtypeStruct((_NC, _NP, _D), jnp.float32)]
    scratch = [
        pltpu.VMEM((_CHUNKS, _K), jnp.int32),
        pltpu.VMEM((_CHUNKS, _K), jnp.int32),
        pltpu.VMEM((_NBUF, _K, _DH), jnp.float32),
        pltpu.VMEM((_ZR, _DH), jnp.float32),
        pltpu.VMEM_SHARED((_NP, _DH), jnp.float32),
    ]
    if with_count:
        out_type = out_type + [
            jax.ShapeDtypeStruct((_NC, _NP, _CW), jnp.float32)]
        scratch = scratch + [
            pltpu.VMEM((_K, _CW), jnp.float32),
            pltpu.VMEM_SHARED((_NP, _CW), jnp.float32),
        ]
    scratch = scratch + [pltpu.SemaphoreType.DMA] * (2 * _NBUF)

    def body(ta_hbm, tb_hbm, edges_hbm, *refs):
        if with_count:
            (out, outc, src_v, dst_v, rows_v, zero_v, acc_s,
             ones_v, accc_s, *sems) = refs
        else:
            out, src_v, dst_v, rows_v, zero_v, acc_s, *sems = refs
        gsem, ssem = sems[:_NBUF], sems[_NBUF:]

        cid = lax.axis_index("c")
        sid = lax.axis_index("s")
        wid = sid * _NC + cid
        stripe = pl.ds(sid * _RPS, _RPS)

        @pl.loop(0, _ZR)
        def _(i):
            for c in range(_DH // _L):
                zero_v.at[pl.ds(i, 1), pl.ds(c * _L, _L)][...] = (
                    jnp.zeros((1, _L), jnp.float32))

        def zero_stripe():
            for blk in range(_RPS // _ZR):
                base = sid * _RPS + blk * _ZR
                pltpu.sync_copy(zero_v, acc_s.at[pl.ds(base, _ZR), :])

        def wait_gather(table, b):
            pltpu.make_async_copy(
                table.at[src_v.at[b]], rows_v.at[b], gsem[b]).wait()

        def wait_scatter(b):
            pltpu.make_async_copy(
                rows_v.at[b], acc_s.at[dst_v.at[b]], ssem[b]).wait()

        def data_pass(table):
            for b in range(_NBUF):
                pltpu.async_copy(table.at[src_v.at[b]], rows_v.at[b],
                                 gsem[b])

            @pl.loop(0, _GRP)
            def _(g):
                for b in range(_NBUF):
                    j = g * _NBUF + b
                    wait_gather(table, b)
                    pltpu.async_copy(rows_v.at[b], acc_s.at[dst_v.at[j]],
                                     ssem[b], add=True)

                    @pl.when(g < _GRP - 1)
                    def _():
                        wait_scatter(b)
                        pltpu.async_copy(table.at[src_v.at[j + _NBUF]],
                                         rows_v.at[b], gsem[b])

            for b in range(_NBUF):
                wait_scatter(b)

        def count_pass():
            @pl.loop(0, _GRP)
            def _(g):
                for b in range(_NBUF):
                    j = g * _NBUF + b

                    @pl.when(g > 0)
                    def _():
                        pltpu.make_async_copy(
                            ones_v, accc_s.at[dst_v.at[b]], ssem[b]).wait()

                    pltpu.async_copy(ones_v, accc_s.at[dst_v.at[j]],
                                     ssem[b], add=True)

            for b in range(_NBUF):
                pltpu.make_async_copy(
                    ones_v, accc_s.at[dst_v.at[b]], ssem[b]).wait()

        zero_stripe()
        if with_count:
            @pl.loop(0, _K)
            def _(i):
                ones_v.at[pl.ds(i, 1), :][...] = jnp.ones((1, _CW),
                                                          jnp.float32)
            for blk in range(_RPS // _ZR):
                base = sid * _RPS + blk * _ZR
                pltpu.sync_copy(zero_v.at[:, pl.ds(0, _CW)],
                                accc_s.at[pl.ds(base, _ZR), :])

        pltpu.sync_copy(edges_hbm.at[0, wid], src_v)
        pltpu.sync_copy(edges_hbm.at[1, wid], dst_v)
        plsc.subcore_barrier()

        if with_count:
            count_pass()

        for pi, table in enumerate([ta_hbm, tb_hbm]):
            data_pass(table)

            plsc.subcore_barrier()
            pltpu.sync_copy(acc_s.at[stripe, :],
                            out.at[cid, stripe, pl.ds(pi * _DH, _DH)])
            if with_count and pi == 0:
                pltpu.sync_copy(accc_s.at[stripe, :],
                                outc.at[cid, stripe, :])
            plsc.subcore_barrier()
            if pi == 0:
                zero_stripe()
                plsc.subcore_barrier()

    return functools.partial(pl.kernel, mesh=_sc_mesh, out_type=out_type,
                             scratch_types=scratch,
                             compiler_params=_sc_params)(body)


_seg_sum_count = _make_seg_sum(with_count=True)
_seg_sum = _make_seg_sum(with_count=False)


def kernel(x, edge_index, W_l1, b1, W_r1, W_l2, b2, W_r2):
    edges = edge_index.reshape(2, _NW, _CHUNKS, _K)

    p1a, p1b, r1 = _dense_pre(x, W_l1, W_r1, b1)
    agg1, cnt = _seg_sum_count(p1a, p1b, edges)
    p2a, p2b, r2 = _dense_mid(agg1, cnt, r1, W_l2, W_r2, b2)
    (agg2,) = _seg_sum(p2a, p2b, edges)
    return _dense_post(agg2, cnt, r2)


# speedup vs baseline: 13.4896x; 1.0557x over previous
import functools

import jax
import jax.numpy as jnp
from jax import lax
from jax.experimental import pallas as pl
from jax.experimental.pallas import tpu as pltpu
from jax.experimental.pallas import tpu_sc as plsc

_NC = 2
_NS = 16
_NW = _NC * _NS
_L = 16

_N = 10000
_E = 320000
_D = 128
_DH = _D // 2

_NP = 10240
_K = 80
_CHUNKS = _E // _NW // _K
_NBUF = 5
_GRP = _CHUNKS // _NBUF
_RPS = _NP // _NS
_ZR = 128
_CW = _L

_ROW_BLK = 1000


def _dot(a, b):
    return lax.dot_general(a, b, (((1,), (0,)), ((), ())),
                           precision=lax.Precision.HIGHEST,
                           preferred_element_type=jnp.float32)


def _dense_pre(x, W_l, W_r, b):
    def body(x_ref, wl_ref, wr_ref, b_ref, p_ref, r_ref):
        xb = x_ref[...]
        p_ref[...] = _dot(xb, wl_ref[...])
        r_ref[...] = _dot(xb, wr_ref[...]) + b_ref[...]

    n = _N
    grid = (n // _ROW_BLK,)
    row = pl.BlockSpec((_ROW_BLK, _D), lambda i: (i, 0))
    return pl.pallas_call(
        body,
        grid=grid,
        in_specs=[
            row,
            pl.BlockSpec((_D, _D), lambda i: (0, 0)),
            pl.BlockSpec((_D, _D), lambda i: (0, 0)),
            pl.BlockSpec((1, _D), lambda i: (0, 0)),
        ],
        out_specs=[row, row],
        out_shape=[jax.ShapeDtypeStruct((n, _D), jnp.float32)] * 2,
    )(x, W_l, W_r, b.reshape(1, _D))


def _merge_agg(a_ref, c_ref, r):
    cnt = c_ref[0, :, 0:1] + c_ref[1, :, 0:1]
    inv = 1.0 / jnp.maximum(cnt, 1.0)
    return (a_ref[0] + a_ref[1]) * inv + r


_agg3 = pl.BlockSpec((_NC, _ROW_BLK, _D), lambda i: (0, i, 0))
_cnt3 = pl.BlockSpec((_NC, _ROW_BLK, _CW), lambda i: (0, i, 0))


def _dense_mid(agg, cnt, r1, W_l, W_r, b):
    def body(a_ref, c_ref, r1_ref, wl_ref, wr_ref, b_ref,
             p_ref, r_ref):
        h = jnp.maximum(_merge_agg(a_ref[...], c_ref[...], r1_ref[...]),
                        0.0)
        p_ref[...] = _dot(h, wl_ref[...])
        r_ref[...] = _dot(h, wr_ref[...]) + b_ref[...]

    n = _N
    grid = (n // _ROW_BLK,)
    row = pl.BlockSpec((_ROW_BLK, _D), lambda i: (i, 0))
    wspec = pl.BlockSpec((_D, _D), lambda i: (0, 0))
    return pl.pallas_call(
        body,
        grid=grid,
        in_specs=[_agg3, _cnt3, row,
                  wspec, wspec, pl.BlockSpec((1, _D), lambda i: (0, 0))],
        out_specs=[row, row],
        out_shape=[jax.ShapeDtypeStruct((n, _D), jnp.float32)] * 2,
    )(agg, cnt, r1, W_l, W_r, b.reshape(1, _D))


def _dense_post(agg, cnt, r2):
    def body(a_ref, c_ref, r2_ref, o_ref):
        o_ref[...] = _merge_agg(a_ref[...], c_ref[...], r2_ref[...])

    n = _N
    grid = (n // _ROW_BLK,)
    row = pl.BlockSpec((_ROW_BLK, _D), lambda i: (i, 0))
    return pl.pallas_call(
        body,
        grid=grid,
        in_specs=[_agg3, _cnt3, row],
        out_specs=row,
        out_shape=jax.ShapeDtypeStruct((n, _D), jnp.float32),
    )(agg, cnt, r2)


_sc_mesh = plsc.VectorSubcoreMesh(core_axis_name="c", subcore_axis_name="s")
_sc_params = pltpu.CompilerParams(use_tc_tiling_on_sc=False)


def _make_seg_sum(with_count):
    out_type = [jax.ShapeDtypeStruct((_NC, _NP, _D), jnp.float32)]
    scratch = [
        pltpu.VMEM((_CHUNKS, _K), jnp.int32),
        pltpu.VMEM((_CHUNKS, _K), jnp.int32),
        pltpu.VMEM((_NBUF, _K, _DH), jnp.float32),
        pltpu.VMEM((_ZR, _DH), jnp.float32),
        pltpu.VMEM_SHARED((_NP, _DH), jnp.float32),
    ]
    if with_count:
        out_type = out_type + [
            jax.ShapeDtypeStruct((_NC, _NP, _CW), jnp.float32)]
        scratch = scratch + [
            pltpu.VMEM((_K, _CW), jnp.float32),
            pltpu.VMEM_SHARED((_NP, _CW), jnp.float32),
        ]
    scratch = scratch + [pltpu.SemaphoreType.DMA] * (2 * _NBUF)

    def body(table_hbm, edges_hbm, *refs):
        if with_count:
            (out, outc, src_v, dst_v, rows_v, zero_v, acc_s,
             ones_v, accc_s, *sems) = refs
        else:
            out, src_v, dst_v, rows_v, zero_v, acc_s, *sems = refs
        gsem, ssem = sems[:_NBUF], sems[_NBUF:]

        cid = lax.axis_index("c")
        sid = lax.axis_index("s")
        wid = sid * _NC + cid
        stripe = pl.ds(sid * _RPS, _RPS)

        @pl.loop(0, _ZR)
        def _(i):
            for c in range(_DH // _L):
                zero_v.at[pl.ds(i, 1), pl.ds(c * _L, _L)][...] = (
                    jnp.zeros((1, _L), jnp.float32))

        def zero_stripe():
            for blk in range(_RPS // _ZR):
                base = sid * _RPS + blk * _ZR
                pltpu.sync_copy(zero_v, acc_s.at[pl.ds(base, _ZR), :])

        def wait_gather(table, b):
            pltpu.make_async_copy(
                table.at[src_v.at[b]], rows_v.at[b], gsem[b]).wait()

        def shift_src(mul, off):
            @pl.loop(0, _CHUNKS)
            def _(i):
                for c in range(_K // _L):
                    sl = src_v.at[pl.ds(i, 1), pl.ds(c * _L, _L)]
                    v = sl[...]
                    sl[...] = v * mul + off

        def wait_scatter(b):
            pltpu.make_async_copy(
                rows_v.at[b], acc_s.at[dst_v.at[b]], ssem[b]).wait()

        def data_pass(table):
            for b in range(_NBUF):
                pltpu.async_copy(table.at[src_v.at[b]], rows_v.at[b],
                                 gsem[b])

            @pl.loop(0, _GRP)
            def _(g):
                for b in range(_NBUF):
                    j = g * _NBUF + b
                    wait_gather(table, b)
                    pltpu.async_copy(rows_v.at[b], acc_s.at[dst_v.at[j]],
                                     ssem[b], add=True)

                    @pl.when(g < _GRP - 1)
                    def _():
                        wait_scatter(b)
                        pltpu.async_copy(table.at[src_v.at[j + _NBUF]],
                                         rows_v.at[b], gsem[b])

            for b in range(_NBUF):
                wait_scatter(b)

        def count_pass():
            @pl.loop(0, _GRP)
            def _(g):
                for b in range(_NBUF):
                    j = g * _NBUF + b

                    @pl.when(g > 0)
                    def _():
                        pltpu.make_async_copy(
                            ones_v, accc_s.at[dst_v.at[b]], ssem[b]).wait()

                    pltpu.async_copy(ones_v, accc_s.at[dst_v.at[j]],
                                     ssem[b], add=True)

            for b in range(_NBUF):
                pltpu.make_async_copy(
                    ones_v, accc_s.at[dst_v.at[b]], ssem[b]).wait()

        zero_stripe()
        if with_count:
            @pl.loop(0, _K)
            def _(i):
                ones_v.at[pl.ds(i, 1), :][...] = jnp.ones((1, _CW),
                                                          jnp.float32)
            for blk in range(_RPS // _ZR):
                base = sid * _RPS + blk * _ZR
                pltpu.sync_copy(zero_v.at[:, pl.ds(0, _CW)],
                                accc_s.at[pl.ds(base, _ZR), :])

        pltpu.sync_copy(edges_hbm.at[0, wid], src_v)
        pltpu.sync_copy(edges_hbm.at[1, wid], dst_v)
        shift_src(2, 0)
        plsc.subcore_barrier()

        if with_count:
            count_pass()

        for pi in range(2):
            if pi == 1:
                shift_src(1, 1)
            data_pass(table_hbm)

            plsc.subcore_barrier()
            pltpu.sync_copy(acc_s.at[stripe, :],
                            out.at[cid, stripe, pl.ds(pi * _DH, _DH)])
            if with_count and pi == 0:
                pltpu.sync_copy(accc_s.at[stripe, :],
                                outc.at[cid, stripe, :])
            plsc.subcore_barrier()
            if pi == 0:
                zero_stripe()
                plsc.subcore_barrier()

    return functools.partial(pl.kernel, mesh=_sc_mesh, out_type=out_type,
                             scratch_types=scratch,
                             compiler_params=_sc_params)(body)


_seg_sum_count = _make_seg_sum(with_count=True)
_seg_sum = _make_seg_sum(with_count=False)


def kernel(x, edge_index, W_l1, b1, W_r1, W_l2, b2, W_r2):
    edges = edge_index.reshape(2, _NW, _CHUNKS, _K)

    p1, r1 = _dense_pre(x, W_l1, W_r1, b1)
    agg1, cnt = _seg_sum_count(p1.reshape(2 * _N, _DH), edges)
    p2, r2 = _dense_mid(agg1, cnt, r1, W_l2, W_r2, b2)
    (agg2,) = _seg_sum(p2.reshape(2 * _N, _DH), edges)
    return _dense_post(agg2, cnt, r2)
